# baseline (speedup 1.0000x reference)
import functools

import jax
import jax.numpy as jnp
from jax import lax
from jax.experimental import pallas as pl
from jax.experimental.pallas import tpu as pltpu
from jax.experimental.pallas import tpu_sc as plsc

N = 10000
D = 128
DH = D // 2
E = 320000
NC = 2
NS = 16
L = 16
CHUNK = 128
CH = 160
NSLOT = 8
HALF = NSLOT // 2
SGC = 40
NSG = CH // SGC
NG = SGC // NSLOT
EW = CH * CHUNK
EPAD = NS * EW
NPAD = 10240
RPT = NPAD // NS
CNTW = 16


def _sc_segsum(tablr, src3, dst3, zrows, zcnt, ocnt, with_cnt):
    mesh = plsc.VectorSubcoreMesh(core_axis_name="c", subcore_axis_name="s")
    out_type = [jax.ShapeDtypeStruct((NC * NPAD, DH), jnp.float32)]
    scratch = (
        [pltpu.VMEM((SGC, CHUNK), jnp.int32)] * 2
        + [pltpu.VMEM((CHUNK, DH), jnp.float32)] * NSLOT
        + [pltpu.VMEM_SHARED((NPAD, DH), jnp.float32)]
        + [pltpu.SemaphoreType.DMA] * (2 * NSLOT)
    )
    if with_cnt:
        out_type.append(jax.ShapeDtypeStruct((NC * NPAD, CNTW), jnp.float32))
        scratch += (
            [pltpu.VMEM((CHUNK, CNTW), jnp.float32)]
            + [pltpu.VMEM_SHARED((NPAD, CNTW), jnp.float32)]
            + [pltpu.SemaphoreType.DMA] * NSLOT
        )

    @functools.partial(
        pl.kernel,
        out_type=tuple(out_type) if with_cnt else out_type[0],
        mesh=mesh,
        scratch_types=tuple(scratch),
        compiler_params=pltpu.CompilerParams(use_tc_tiling_on_sc=False),
    )
    def k(tab_h, src_h, dst_h, zrows_h, zcnt_h, ocnt_h, *rest):
        outs = 2 if with_cnt else 1
        if with_cnt:
            ssum_out, cnt_out = rest[0], rest[1]
        else:
            ssum_out = rest[0]
        srcall_v, dstall_v = rest[outs], rest[outs + 1]
        rows = list(rest[outs + 2:outs + 2 + NSLOT])
        acc_s = rest[outs + 2 + NSLOT]
        gsem = list(rest[outs + 3 + NSLOT:outs + 3 + 2 * NSLOT])
        ssem = list(rest[outs + 3 + 2 * NSLOT:outs + 3 + 3 * NSLOT])
        if with_cnt:
            cnt_v = rest[outs + 3 + 3 * NSLOT]
            cacc_s = rest[outs + 4 + 3 * NSLOT]
            csem = list(rest[outs + 5 + 3 * NSLOT:outs + 5 + 4 * NSLOT])
        cid = lax.axis_index("c")
        sid = lax.axis_index("s")
        w = cid * NS + sid
        sb = pl.multiple_of(w * CH, CH)
        db = pl.multiple_of(sid * CH, CH)

        pltpu.sync_copy(zrows_h, rows[0])
        if with_cnt:
            pltpu.sync_copy(zcnt_h, cnt_v)
        for kk in range(RPT // CHUNK):
            r0 = sid * RPT + kk * CHUNK
            pltpu.sync_copy(rows[0], acc_s.at[pl.ds(r0, CHUNK)])
            if with_cnt:
                pltpu.sync_copy(cnt_v, cacc_s.at[pl.ds(r0, CHUNK)])
        if with_cnt:
            pltpu.sync_copy(ocnt_h, cnt_v)

        def gather(lc, j):
            pltpu.async_copy(tab_h.at[srcall_v.at[lc]], rows[j], gsem[j])

        def wait_gather(lc, j):
            pltpu.make_async_copy(
                tab_h.at[srcall_v.at[lc]], rows[j], gsem[j]).wait()

        def drain_refill(g, lc0, j):
            pltpu.make_async_copy(
                rows[j], acc_s.at[dstall_v.at[lc0 + j]], ssem[j]).wait()
            if with_cnt:
                pltpu.make_async_copy(
                    cnt_v, cacc_s.at[dstall_v.at[lc0 + j]], csem[j]).wait()

            @pl.when(g < NG - 1)
            def _():
                gather(lc0 + j + NSLOT, j)

        plsc.subcore_barrier()

        @pl.loop(0, NSG)
        def supergroup(sg):
            pltpu.sync_copy(src_h.at[pl.ds(sb + sg * SGC, SGC)], srcall_v)
            pltpu.sync_copy(dst_h.at[pl.ds(db + sg * SGC, SGC)], dstall_v)
            for j in range(NSLOT):
                gather(j, j)

            @pl.loop(0, NG)
            def group(g):
                lc0 = g * NSLOT
                for j in range(NSLOT):
                    lc = lc0 + j
                    wait_gather(lc, j)
                    pltpu.async_copy(
                        rows[j], acc_s.at[dstall_v.at[lc]], ssem[j],
                        add=True)
                    if with_cnt:
                        pltpu.async_copy(
                            cnt_v, cacc_s.at[dstall_v.at[lc]], csem[j],
                            add=True)
                    if j >= HALF:
                        drain_refill(g, lc0, j - HALF)
                for j in range(HALF, NSLOT):
                    drain_refill(g, lc0, j)

        plsc.subcore_barrier()

        obase = cid * NPAD + sid * RPT
        for kk in range(RPT // CHUNK):
            r0 = sid * RPT + kk * CHUNK
            q0 = pl.multiple_of(obase + kk * CHUNK, CHUNK)
            pltpu.sync_copy(acc_s.at[pl.ds(r0, CHUNK)], rows[0])
            pltpu.sync_copy(rows[0], ssum_out.at[pl.ds(q0, CHUNK)])
            if with_cnt:
                pltpu.sync_copy(cacc_s.at[pl.ds(r0, CHUNK)], cnt_v)
                pltpu.sync_copy(cnt_v, cnt_out.at[pl.ds(q0, CHUNK)])

    return k(tablr, src3, dst3, zrows, zcnt, ocnt)


def _tc_layer(ssum, cnt, h, WlT, WrT, b, relu):
    R = 400

    def body(ssum_ref, cnt_ref, h_ref, wl_ref, wr_ref, b_ref, o_ref):
        s = jnp.concatenate([ssum_ref[0], ssum_ref[1]], axis=1)
        c = cnt_ref[0, :, 0:1]
        m = s / jnp.maximum(c, 1.0)
        o = jnp.dot(m, wl_ref[...], preferred_element_type=jnp.float32)
        o = o + jnp.dot(h_ref[...], wr_ref[...],
                        preferred_element_type=jnp.float32)
        o = o + b_ref[...]
        if relu:
            o = jnp.maximum(o, 0.0)
        o_ref[...] = o

    return pl.pallas_call(
        body,
        grid=(N // R,),
        in_specs=[
            pl.BlockSpec((NC, R, DH), lambda i: (0, i, 0)),
            pl.BlockSpec((1, R, CNTW), lambda i: (0, i, 0)),
            pl.BlockSpec((R, D), lambda i: (i, 0)),
            pl.BlockSpec((D, D), lambda i: (0, 0)),
            pl.BlockSpec((D, D), lambda i: (0, 0)),
            pl.BlockSpec((1, D), lambda i: (0, 0)),
        ],
        out_specs=pl.BlockSpec((R, D), lambda i: (i, 0)),
        out_shape=jax.ShapeDtypeStruct((N, D), jnp.float32),
    )(ssum, cnt, h, WlT, WrT, b.reshape(1, D))


def kernel(x, edge_index, Wl1, bl1, Wr1, Wl2, bl2, Wr2):
    src = edge_index[0]
    dst = edge_index[1]
    pad = EPAD - E
    srcp = jnp.concatenate([src, jnp.zeros((pad,), jnp.int32)])
    dstp = jnp.concatenate([dst, jnp.full((pad,), N, jnp.int32)])
    src3 = jnp.concatenate([2 * srcp, 2 * srcp + 1]).reshape(
        NC * NS * CH, CHUNK)
    dst3 = dstp.reshape(NS * CH, CHUNK)
    zrows = jnp.zeros((CHUNK, DH), jnp.float32)
    zcnt = jnp.zeros((CHUNK, CNTW), jnp.float32)
    ocnt = jnp.ones((CHUNK, CNTW), jnp.float32)

    ssum1, cnt = _sc_segsum(x.reshape(NC * N, DH), src3, dst3, zrows, zcnt,
                            ocnt, with_cnt=True)
    ssum1 = ssum1.reshape(NC, NPAD, DH)
    cnt = cnt.reshape(NC, NPAD, CNTW)
    h = _tc_layer(ssum1, cnt, x, Wl1.T, Wr1.T, bl1, relu=True)
    ssum2 = _sc_segsum(h.reshape(NC * N, DH), src3, dst3, zrows, zcnt,
                       ocnt, with_cnt=False)
    ssum2 = ssum2.reshape(NC, NPAD, DH)
    return _tc_layer(ssum2, cnt, h, Wl2.T, Wr2.T, bl2, relu=False)


# speedup vs baseline: 1.2048x; 1.2048x over previous
import functools

import jax
import jax.numpy as jnp
from jax import lax
from jax.experimental import pallas as pl
from jax.experimental.pallas import tpu as pltpu
from jax.experimental.pallas import tpu_sc as plsc

N = 10000
D = 128
DH = D // 2
E = 320000
NC = 2
NS = 16
L = 16
CHUNK = 128
CH = 160
NSLOT = 8
HALF = NSLOT // 2
SGC = 40
NSG = CH // SGC
NG = SGC // NSLOT
EW = CH * CHUNK
EPAD = NS * EW
NPAD = 10240
RPT = NPAD // NS
CNTW = 16


def _sc_segsum(tablr, src3, dst3, zrows, zcnt, ocnt, with_cnt):
    mesh = plsc.VectorSubcoreMesh(core_axis_name="c", subcore_axis_name="s")
    out_type = [jax.ShapeDtypeStruct((NC * NPAD, DH), jnp.float32)]
    scratch = (
        [pltpu.VMEM((SGC, CHUNK), jnp.int32)] * 2
        + [pltpu.VMEM((CHUNK, DH), jnp.float32)] * NSLOT
        + [pltpu.VMEM_SHARED((NPAD, DH), jnp.float32)]
        + [pltpu.SemaphoreType.DMA] * (2 * NSLOT)
    )
    if with_cnt:
        out_type.append(jax.ShapeDtypeStruct((NC * NPAD, CNTW), jnp.float32))
        scratch += (
            [pltpu.VMEM((CHUNK, CNTW), jnp.float32)]
            + [pltpu.VMEM_SHARED((NPAD, CNTW), jnp.float32)]
            + [pltpu.SemaphoreType.DMA] * NSLOT
        )

    @functools.partial(
        pl.kernel,
        out_type=tuple(out_type) if with_cnt else out_type[0],
        mesh=mesh,
        scratch_types=tuple(scratch),
        compiler_params=pltpu.CompilerParams(use_tc_tiling_on_sc=False),
    )
    def k(tab_h, src_h, dst_h, zrows_h, zcnt_h, ocnt_h, *rest):
        outs = 2 if with_cnt else 1
        if with_cnt:
            ssum_out, cnt_out = rest[0], rest[1]
        else:
            ssum_out = rest[0]
        srcall_v, dstall_v = rest[outs], rest[outs + 1]
        rows = list(rest[outs + 2:outs + 2 + NSLOT])
        acc_s = rest[outs + 2 + NSLOT]
        gsem = list(rest[outs + 3 + NSLOT:outs + 3 + 2 * NSLOT])
        ssem = list(rest[outs + 3 + 2 * NSLOT:outs + 3 + 3 * NSLOT])
        if with_cnt:
            cnt_v = rest[outs + 3 + 3 * NSLOT]
            cacc_s = rest[outs + 4 + 3 * NSLOT]
            csem = list(rest[outs + 5 + 3 * NSLOT:outs + 5 + 4 * NSLOT])
        cid = lax.axis_index("c")
        sid = lax.axis_index("s")
        w = cid * NS + sid
        sb = pl.multiple_of(w * CH, CH)
        db = pl.multiple_of(sid * CH, CH)

        pltpu.sync_copy(zrows_h, rows[0])
        if with_cnt:
            pltpu.sync_copy(zcnt_h, cnt_v)
        for kk in range(RPT // CHUNK):
            r0 = sid * RPT + kk * CHUNK
            pltpu.sync_copy(rows[0], acc_s.at[pl.ds(r0, CHUNK)])
            if with_cnt:
                pltpu.sync_copy(cnt_v, cacc_s.at[pl.ds(r0, CHUNK)])
        if with_cnt:
            pltpu.sync_copy(ocnt_h, cnt_v)

        def gather(lc, j):
            pltpu.async_copy(tab_h.at[srcall_v.at[lc]], rows[j], gsem[j])

        def wait_gather(lc, j):
            pltpu.make_async_copy(
                tab_h.at[srcall_v.at[lc]], rows[j], gsem[j]).wait()

        def drain_refill(g, lc0, j):
            pltpu.make_async_copy(
                rows[j], acc_s.at[dstall_v.at[lc0 + j]], ssem[j]).wait()
            if with_cnt:
                pltpu.make_async_copy(
                    cnt_v, cacc_s.at[dstall_v.at[lc0 + j]], csem[j]).wait()

            @pl.when(g < NG - 1)
            def _():
                gather(lc0 + j + NSLOT, j)

        plsc.subcore_barrier()

        @pl.loop(0, NSG)
        def supergroup(sg):
            pltpu.sync_copy(src_h.at[pl.ds(sb + sg * SGC, SGC)], srcall_v)
            pltpu.sync_copy(dst_h.at[pl.ds(db + sg * SGC, SGC)], dstall_v)
            for j in range(NSLOT):
                gather(j, j)

            @pl.loop(0, NG)
            def group(g):
                lc0 = g * NSLOT
                for j in range(NSLOT):
                    lc = lc0 + j
                    wait_gather(lc, j)
                    pltpu.async_copy(
                        rows[j], acc_s.at[dstall_v.at[lc]], ssem[j],
                        add=True)
                    if with_cnt:
                        pltpu.async_copy(
                            cnt_v, cacc_s.at[dstall_v.at[lc]], csem[j],
                            add=True)
                    if j >= HALF:
                        drain_refill(g, lc0, j - HALF)
                for j in range(HALF, NSLOT):
                    drain_refill(g, lc0, j)

        plsc.subcore_barrier()

        obase = cid * NPAD + sid * RPT
        for kk in range(RPT // CHUNK):
            r0 = sid * RPT + kk * CHUNK
            q0 = pl.multiple_of(obase + kk * CHUNK, CHUNK)
            pltpu.sync_copy(acc_s.at[pl.ds(r0, CHUNK)], rows[0])
            pltpu.sync_copy(rows[0], ssum_out.at[pl.ds(q0, CHUNK)])
            if with_cnt:
                pltpu.sync_copy(cacc_s.at[pl.ds(r0, CHUNK)], cnt_v)
                pltpu.sync_copy(cnt_v, cnt_out.at[pl.ds(q0, CHUNK)])

    return k(tablr, src3, dst3, zrows, zcnt, ocnt)


def _tc_layer(ssum, cnt, h, WlT, WrT, b, relu, emit_split=False):
    R = 400

    def body(ssum_ref, cnt_ref, h_ref, wl_ref, wr_ref, b_ref, *outs):
        s = jnp.concatenate([ssum_ref[0], ssum_ref[1]], axis=1)
        c = cnt_ref[0, :, 0:1]
        m = s / jnp.maximum(c, 1.0)
        o = jnp.dot(m, wl_ref[...], preferred_element_type=jnp.float32)
        o = o + jnp.dot(h_ref[...], wr_ref[...],
                        preferred_element_type=jnp.float32)
        o = o + b_ref[...]
        if relu:
            o = jnp.maximum(o, 0.0)
        outs[0][...] = o
        if emit_split:
            outs[1][0] = o[:, :DH]
            outs[1][1] = o[:, DH:]

    out_shape = [jax.ShapeDtypeStruct((N, D), jnp.float32)]
    out_specs = [pl.BlockSpec((R, D), lambda i: (i, 0))]
    if emit_split:
        out_shape.append(jax.ShapeDtypeStruct((NC, N, DH), jnp.float32))
        out_specs.append(pl.BlockSpec((NC, R, DH), lambda i: (0, i, 0)))

    return pl.pallas_call(
        body,
        grid=(N // R,),
        in_specs=[
            pl.BlockSpec((NC, R, DH), lambda i: (0, i, 0)),
            pl.BlockSpec((1, R, CNTW), lambda i: (0, i, 0)),
            pl.BlockSpec((R, D), lambda i: (i, 0)),
            pl.BlockSpec((D, D), lambda i: (0, 0)),
            pl.BlockSpec((D, D), lambda i: (0, 0)),
            pl.BlockSpec((1, D), lambda i: (0, 0)),
        ],
        out_specs=out_specs if emit_split else out_specs[0],
        out_shape=out_shape if emit_split else out_shape[0],
    )(ssum, cnt, h, WlT, WrT, b.reshape(1, D))


def kernel(x, edge_index, Wl1, bl1, Wr1, Wl2, bl2, Wr2):
    src = edge_index[0]
    dst = edge_index[1]
    pad = EPAD - E
    srcp = jnp.concatenate([src, jnp.zeros((pad,), jnp.int32)])
    dstp = jnp.concatenate([dst, jnp.full((pad,), N, jnp.int32)])
    src3 = jnp.concatenate([srcp, srcp + N]).reshape(NC * NS * CH, CHUNK)
    dst3 = dstp.reshape(NS * CH, CHUNK)
    zrows = jnp.zeros((CHUNK, DH), jnp.float32)
    zcnt = jnp.zeros((CHUNK, CNTW), jnp.float32)
    ocnt = jnp.ones((CHUNK, CNTW), jnp.float32)

    xlr = jnp.concatenate([x[:, :DH], x[:, DH:]], axis=0)
    ssum1, cnt = _sc_segsum(xlr, src3, dst3, zrows, zcnt,
                            ocnt, with_cnt=True)
    ssum1 = ssum1.reshape(NC, NPAD, DH)
    cnt = cnt.reshape(NC, NPAD, CNTW)
    h, hsplit = _tc_layer(ssum1, cnt, x, Wl1.T, Wr1.T, bl1, relu=True,
                          emit_split=True)
    ssum2 = _sc_segsum(hsplit.reshape(NC * N, DH), src3, dst3, zrows, zcnt,
                       ocnt, with_cnt=False)
    ssum2 = ssum2.reshape(NC, NPAD, DH)
    return _tc_layer(ssum2, cnt, h, Wl2.T, Wr2.T, bl2, relu=False)
